# Initial kernel scaffold; baseline (speedup 1.0000x reference)
import jax
import jax.numpy as jnp
from jax.experimental import pallas as pl


def kernel(x, atomic_numbers, scales):
    raise NotImplementedError("write your pallas kernel here")


# speedup vs baseline: 5.4153x; 5.4153x over previous
import jax
import jax.numpy as jnp
from jax.experimental import pallas as pl

_R = 2000


def _body(an_ref, tab_ref, x_ref, o_ref):
    an = an_ref[...]
    rs = jax.lax.rsqrt(tab_ref[...])
    lane = jax.lax.broadcasted_iota(jnp.int32, (an.shape[0], 128), 1)
    onehot = (lane == an).astype(jnp.float32)
    s = jax.lax.dot_general(
        onehot, rs,
        dimension_numbers=(((1,), (1,)), ((), ())),
        preferred_element_type=jnp.float32,
    )
    o_ref[...] = x_ref[...] * s


def kernel(x, atomic_numbers, scales):
    n, d = x.shape
    an = atomic_numbers.astype(jnp.int32).reshape(n, 1)
    tab = jnp.concatenate(
        [scales[:, 0], jnp.ones((128 - scales.shape[0],), jnp.float32)]
    ).reshape(1, 128)
    grid = (n // _R,)
    return pl.pallas_call(
        _body,
        grid=grid,
        in_specs=[
            pl.BlockSpec((_R, 1), lambda i: (i, 0)),
            pl.BlockSpec((1, 128), lambda i: (0, 0)),
            pl.BlockSpec((_R, d), lambda i: (i, 0)),
        ],
        out_specs=pl.BlockSpec((_R, d), lambda i: (i, 0)),
        out_shape=jax.ShapeDtypeStruct((n, d), x.dtype),
    )(an, tab, x)


# speedup vs baseline: 9.2793x; 1.7135x over previous
import jax
import jax.numpy as jnp
from jax.experimental import pallas as pl

_R = 2000


def _body(an_ref, tab_ref, x_ref, o_ref):
    an = an_ref[...].reshape(_R, 1)
    rs = jax.lax.rsqrt(tab_ref[...])
    lane = jax.lax.broadcasted_iota(jnp.int32, (_R, 128), 1)
    onehot = (lane == an).astype(jnp.float32)
    s = jax.lax.dot_general(
        onehot, rs,
        dimension_numbers=(((1,), (1,)), ((), ())),
        preferred_element_type=jnp.float32,
    )
    o_ref[...] = x_ref[...] * s


def kernel(x, atomic_numbers, scales):
    n, d = x.shape
    nb = n // _R
    an = atomic_numbers.astype(jnp.int32).reshape(nb, 1, _R)
    tab = jnp.concatenate(
        [scales[:, 0], jnp.ones((128 - scales.shape[0],), jnp.float32)]
    ).reshape(1, 128)
    return pl.pallas_call(
        _body,
        grid=(nb,),
        in_specs=[
            pl.BlockSpec((1, 1, _R), lambda i: (i, 0, 0)),
            pl.BlockSpec((1, 128), lambda i: (0, 0)),
            pl.BlockSpec((_R, d), lambda i: (i, 0)),
        ],
        out_specs=pl.BlockSpec((_R, d), lambda i: (i, 0)),
        out_shape=jax.ShapeDtypeStruct((n, d), x.dtype),
    )(an, tab, x)


# speedup vs baseline: 12.4512x; 1.3418x over previous
import jax
import jax.numpy as jnp
from jax.experimental import pallas as pl

_R = 5000


def _body(an_ref, tab_ref, x_ref, o_ref):
    an = an_ref[...].reshape(_R, 1)
    rs = jax.lax.rsqrt(tab_ref[...])
    lane = jax.lax.broadcasted_iota(jnp.int32, (_R, 128), 1)
    onehot = (lane == an).astype(jnp.float32)
    s = jax.lax.dot_general(
        onehot, rs,
        dimension_numbers=(((1,), (1,)), ((), ())),
        preferred_element_type=jnp.float32,
    )
    o_ref[...] = x_ref[...] * s


def kernel(x, atomic_numbers, scales):
    n, d = x.shape
    nb = n // _R
    an = atomic_numbers.astype(jnp.int32).reshape(nb, 1, _R)
    tab = jnp.concatenate(
        [scales[:, 0], jnp.ones((128 - scales.shape[0],), jnp.float32)]
    ).reshape(1, 128)
    return pl.pallas_call(
        _body,
        grid=(nb,),
        in_specs=[
            pl.BlockSpec((1, 1, _R), lambda i: (i, 0, 0)),
            pl.BlockSpec((1, 128), lambda i: (0, 0)),
            pl.BlockSpec((_R, d), lambda i: (i, 0)),
        ],
        out_specs=pl.BlockSpec((_R, d), lambda i: (i, 0)),
        out_shape=jax.ShapeDtypeStruct((n, d), x.dtype),
    )(an, tab, x)


# speedup vs baseline: 13.7111x; 1.1012x over previous
import jax
import jax.numpy as jnp
from jax.experimental import pallas as pl

_R = 10000


def _body(an_ref, tab_ref, x_ref, o_ref):
    an = an_ref[...].reshape(_R, 1)
    rs = jax.lax.rsqrt(tab_ref[...])
    lane = jax.lax.broadcasted_iota(jnp.int32, (_R, 128), 1)
    onehot = (lane == an).astype(jnp.float32)
    s = jax.lax.dot_general(
        onehot, rs,
        dimension_numbers=(((1,), (1,)), ((), ())),
        preferred_element_type=jnp.float32,
    )
    o_ref[...] = x_ref[...] * s


def kernel(x, atomic_numbers, scales):
    n, d = x.shape
    nb = n // _R
    an = atomic_numbers.astype(jnp.int32).reshape(nb, 1, _R)
    tab = jnp.concatenate(
        [scales[:, 0], jnp.ones((128 - scales.shape[0],), jnp.float32)]
    ).reshape(1, 128)
    return pl.pallas_call(
        _body,
        grid=(nb,),
        in_specs=[
            pl.BlockSpec((1, 1, _R), lambda i: (i, 0, 0)),
            pl.BlockSpec((1, 128), lambda i: (0, 0)),
            pl.BlockSpec((_R, d), lambda i: (i, 0)),
        ],
        out_specs=pl.BlockSpec((_R, d), lambda i: (i, 0)),
        out_shape=jax.ShapeDtypeStruct((n, d), x.dtype),
    )(an, tab, x)
